# baseline (speedup 1.0000x reference)
import functools

import jax
import jax.numpy as jnp
from jax import lax
from jax.experimental import pallas as pl
from jax.experimental.pallas import tpu as pltpu
from jax.experimental.pallas import tpu_sc as plsc

B = 16384
L = 50
D = 64
OUT = 64
DP = D // 2

NC = 2
NS = 16
NW = NC * NS
BAGS_PER_W = B // NW
GI = 100
CH = 8
NG = CH * L // GI
NCHUNK = BAGS_PER_W // CH
NPAIR = NCHUNK // 2
IROWS_PER_W = BAGS_PER_W * L // GI


def _sc_pool_kernel(x2d_hbm, table_hbm, out_hbm, idx_v, rows_v, pooled_v,
                    sem0, sem1):
    wid = lax.axis_index("s") * NC + lax.axis_index("c")
    bag0 = pl.multiple_of(wid * BAGS_PER_W, BAGS_PER_W)
    irow0 = pl.multiple_of(wid * IROWS_PER_W, IROWS_PER_W)

    pltpu.sync_copy(x2d_hbm.at[pl.ds(irow0, IROWS_PER_W)], idx_v)

    def gather(c, buf, sem):
        for g in range(NG):
            pltpu.async_copy(
                table_hbm.at[idx_v.at[c * NG + g]],
                rows_v.at[buf, g],
                sem,
            )

    def drain(c, buf, sem):
        for g in range(NG):
            pltpu.make_async_copy(
                table_hbm.at[idx_v.at[c * NG + g]],
                rows_v.at[buf, g],
                sem,
            ).wait()

    def pool(c, buf):
        def bag_body(j, carry2):
            g = j // 2
            off = (j % 2) * L

            def l_body(l, acc):
                a0, b0, a1, b1 = acc
                w0 = rows_v[buf, g, off + l, pl.ds(0, 16)]
                w1 = rows_v[buf, g, off + l, pl.ds(16, 16)]
                e0, o0 = plsc.unpack(
                    plsc.bitcast(w0, jnp.bfloat16),
                    format=plsc.PackFormat.INTERLEAVED,
                    preferred_element_type=jnp.float32,
                )
                e1, o1 = plsc.unpack(
                    plsc.bitcast(w1, jnp.bfloat16),
                    format=plsc.PackFormat.INTERLEAVED,
                    preferred_element_type=jnp.float32,
                )
                return (a0 + e0, b0 + o0, a1 + e1, b1 + o1)

            z = jnp.zeros((16,), jnp.float32)
            a0, b0, a1, b1 = lax.fori_loop(0, L, l_body, (z, z, z, z))
            slot = c * CH + j
            inv = 1.0 / L
            pooled_v[slot, pl.ds(0, 16)] = a0 * inv
            pooled_v[slot, pl.ds(32, 16)] = b0 * inv
            pooled_v[slot, pl.ds(16, 16)] = a1 * inv
            pooled_v[slot, pl.ds(48, 16)] = b1 * inv
            return carry2

        lax.fori_loop(0, CH, bag_body, 0)

    gather(0, 0, sem0)

    def pair_body(t, carry):
        c0 = t * 2
        c1 = c0 + 1
        gather(c1, 1, sem1)
        drain(c0, 0, sem0)
        pool(c0, 0)

        @pl.when(t < NPAIR - 1)
        def _():
            gather(c0 + 2, 0, sem0)

        drain(c1, 1, sem1)
        pool(c1, 1)
        return carry

    lax.fori_loop(0, NPAIR, pair_body, 0)
    pltpu.sync_copy(pooled_v, out_hbm.at[pl.ds(bag0, BAGS_PER_W)])


_sc_pool = functools.partial(
    pl.kernel,
    mesh=plsc.VectorSubcoreMesh(core_axis_name="c", subcore_axis_name="s"),
    out_type=jax.ShapeDtypeStruct((B, D), jnp.float32),
    scratch_types=[
        pltpu.VMEM((IROWS_PER_W, GI), jnp.int32),
        pltpu.VMEM((2, NG, GI, DP), jnp.uint32),
        pltpu.VMEM((BAGS_PER_W, D), jnp.float32),
        pltpu.SemaphoreType.DMA,
        pltpu.SemaphoreType.DMA,
    ],
    compiler_params=pltpu.CompilerParams(use_tc_tiling_on_sc=False,
                                         needs_layout_passes=False),
)(_sc_pool_kernel)


VBLK = 8192


def _tc_pack_kernel(t_ref, o_ref):
    t32 = lax.bitcast_convert_type(t_ref[...], jnp.uint32)
    r = t32 + jnp.uint32(0x8000)
    lo = r[0:DP, :] >> 16
    hi = r[DP:D, :] & jnp.uint32(0xFFFF0000)
    o_ref[...] = lo | hi


def _tc_pack(table_t):
    n = table_t.shape[1]
    return pl.pallas_call(
        _tc_pack_kernel,
        grid=(pl.cdiv(n, VBLK),),
        in_specs=[pl.BlockSpec((D, VBLK), lambda i: (0, i))],
        out_specs=pl.BlockSpec((DP, VBLK), lambda i: (0, i)),
        out_shape=jax.ShapeDtypeStruct((DP, n), jnp.uint32),
    )(table_t)


BLK = 512


def _tc_head_kernel(p_ref, wt_ref, b_ref, o_ref):
    y = jnp.dot(p_ref[...], wt_ref[...], preferred_element_type=jnp.float32)
    y = y + b_ref[...]
    y = y - jnp.max(y, axis=1, keepdims=True)
    e = jnp.exp(y)
    o_ref[...] = e / jnp.sum(e, axis=1, keepdims=True)


def _tc_head(pooled, wt, b2):
    return pl.pallas_call(
        _tc_head_kernel,
        grid=(B // BLK,),
        in_specs=[
            pl.BlockSpec((BLK, D), lambda i: (i, 0)),
            pl.BlockSpec((D, OUT), lambda i: (0, 0)),
            pl.BlockSpec((1, OUT), lambda i: (0, 0)),
        ],
        out_specs=pl.BlockSpec((BLK, OUT), lambda i: (i, 0)),
        out_shape=jax.ShapeDtypeStruct((B, OUT), jnp.float32),
    )(pooled, wt, b2)


def kernel(x, emb_table, W, b):
    table_packed = _tc_pack(emb_table.T).T
    x2d = x.astype(jnp.int32).reshape(B * L // GI, GI)
    pooled = _sc_pool(x2d, table_packed)
    return _tc_head(pooled, W.T, b.reshape(1, OUT))


# speedup vs baseline: 1.2202x; 1.2202x over previous
import functools

import jax
import jax.numpy as jnp
from jax import lax
from jax.experimental import pallas as pl
from jax.experimental.pallas import tpu as pltpu
from jax.experimental.pallas import tpu_sc as plsc

B = 16384
L = 50
D = 64
OUT = 64

NC = 2
NS = 16
NW = NC * NS
BAGS_PER_W = B // NW
CH = 8
NCHUNK = BAGS_PER_W // CH
NPAIR = NCHUNK // 2
NVEC = D // 16


def _sc_pool_kernel(x_hbm, table_hbm, out_hbm, idx_v, rows_v, pooled_v,
                    sem0, sem1):
    wid = lax.axis_index("s") * NC + lax.axis_index("c")
    bag0 = pl.multiple_of(wid * BAGS_PER_W, BAGS_PER_W)

    pltpu.sync_copy(x_hbm.at[pl.ds(bag0, BAGS_PER_W)], idx_v)

    def gather(c, buf, sem):
        for j in range(CH):
            pltpu.async_copy(
                table_hbm.at[idx_v.at[c * CH + j]],
                rows_v.at[buf, j],
                sem,
            )

    def drain(c, buf, sem):
        for j in range(CH):
            pltpu.make_async_copy(
                table_hbm.at[idx_v.at[c * CH + j]],
                rows_v.at[buf, j],
                sem,
            ).wait()

    def pool(c, buf):
        def bag_body(j, carry2):
            def l_body(l, acc):
                return tuple(
                    acc[k] + rows_v[buf, j, l, pl.ds(16 * k, 16)]
                    for k in range(NVEC)
                )

            acc = lax.fori_loop(
                0, L, l_body,
                tuple(jnp.zeros((16,), jnp.float32) for _ in range(NVEC)),
            )
            for k in range(NVEC):
                pooled_v[c * CH + j, pl.ds(16 * k, 16)] = acc[k] * (1.0 / L)
            return carry2

        lax.fori_loop(0, CH, bag_body, 0)

    gather(0, 0, sem0)

    def pair_body(t, carry):
        c0 = t * 2
        c1 = c0 + 1
        gather(c1, 1, sem1)
        drain(c0, 0, sem0)
        pool(c0, 0)

        @pl.when(t < NPAIR - 1)
        def _():
            gather(c0 + 2, 0, sem0)

        drain(c1, 1, sem1)
        pool(c1, 1)
        return carry

    lax.fori_loop(0, NPAIR, pair_body, 0)
    pltpu.sync_copy(pooled_v, out_hbm.at[pl.ds(bag0, BAGS_PER_W)])


_sc_pool = functools.partial(
    pl.kernel,
    mesh=plsc.VectorSubcoreMesh(core_axis_name="c", subcore_axis_name="s"),
    out_type=jax.ShapeDtypeStruct((B, D), jnp.float32),
    scratch_types=[
        pltpu.VMEM((BAGS_PER_W, L), jnp.int32),
        pltpu.VMEM((2, CH, L, D), jnp.float32),
        pltpu.VMEM((BAGS_PER_W, D), jnp.float32),
        pltpu.SemaphoreType.DMA,
        pltpu.SemaphoreType.DMA,
    ],
    compiler_params=pltpu.CompilerParams(use_tc_tiling_on_sc=False),
)(_sc_pool_kernel)


BLK = 512


def _tc_head_kernel(p_ref, wt_ref, b_ref, o_ref):
    y = jnp.dot(p_ref[...], wt_ref[...], preferred_element_type=jnp.float32)
    y = y + b_ref[...]
    y = y - jnp.max(y, axis=1, keepdims=True)
    e = jnp.exp(y)
    o_ref[...] = e / jnp.sum(e, axis=1, keepdims=True)


def _tc_head(pooled, wt, b2):
    return pl.pallas_call(
        _tc_head_kernel,
        grid=(B // BLK,),
        in_specs=[
            pl.BlockSpec((BLK, D), lambda i: (i, 0)),
            pl.BlockSpec((D, OUT), lambda i: (0, 0)),
            pl.BlockSpec((1, OUT), lambda i: (0, 0)),
        ],
        out_specs=pl.BlockSpec((BLK, OUT), lambda i: (i, 0)),
        out_shape=jax.ShapeDtypeStruct((B, OUT), jnp.float32),
    )(pooled, wt, b2)


def kernel(x, emb_table, W, b):
    pooled = _sc_pool(x.astype(jnp.int32), emb_table)
    return _tc_head(pooled, W.T, b.reshape(1, OUT))
